# baseline (speedup 1.0000x reference)
import functools

import jax
import jax.numpy as jnp
from jax.experimental import pallas as pl
from jax.experimental.pallas import tpu as pltpu


def _angle(hi, hr):
    m = (hr * hr + hi * hi) > 1e-12
    hr_s = jnp.where(m, hr, 1.0)
    hi_s = jnp.where(m, hi, 0.0)
    return jnp.where(m, jnp.arctan2(hi_s, hr_s), 0.0)


def _lin0_body(x_ref, w_ref, b_ref, h_ref):
    h_ref[...] = jnp.maximum(
        jnp.dot(x_ref[...], w_ref[...], preferred_element_type=jnp.float32)
        + b_ref[...], 0.0).astype(jnp.bfloat16)


def _layer_body(*refs, final, bn, nblk, h_dim, n_cls):
    if final:
        (adj_ref, h_ref, m_ref, wc_ref, lab_ref, msk_ref, mo_ref, bo_ref,
         hr_ref, hi_ref, ang_ref, nrm_ref, oo_ref, sep_ref, sup_ref,
         cm_acc, cnt_acc, sup_acc, msk_acc) = refs
    else:
        (adj_ref, h_ref, m_ref, wc_ref, lab_ref, msk_ref,
         hout_ref, adjb_ref, sep_ref, sup_ref,
         cm_acc, cnt_acc, sup_acc, msk_acc) = refs
    i = pl.program_id(0)

    @pl.when(i == 0)
    def _init():
        cm_acc[...] = jnp.zeros_like(cm_acc)
        cnt_acc[...] = jnp.zeros_like(cnt_acc)
        sup_acc[...] = jnp.zeros_like(sup_acc)
        msk_acc[...] = jnp.zeros_like(msk_acc)

    if final:
        adjb = adj_ref[...]
    else:
        adjb = adj_ref[...].astype(jnp.bfloat16)
        adjb_ref[...] = adjb
    a = jnp.dot(adjb, h_ref[...], preferred_element_type=jnp.float32)
    n = jnp.dot(a, m_ref[...], preferred_element_type=jnp.float32)
    nr = n[:, :h_dim]
    ni = n[:, h_dim:]

    logits = jnp.dot(n, wc_ref[...], preferred_element_type=jnp.float32)
    mx = jnp.max(logits, axis=-1, keepdims=True)
    lse = mx + jnp.log(jnp.sum(jnp.exp(logits - mx), axis=-1, keepdims=True))
    logp = logits - lse
    lab = lab_ref[...]
    classes = jax.lax.broadcasted_iota(jnp.int32, (bn, n_cls), 1)
    onehot = (lab == classes).astype(jnp.float32)
    maskf = msk_ref[...]
    picked = jnp.sum(onehot * logp, axis=-1, keepdims=True)
    sup_acc[...] = sup_acc[...] + jnp.sum(picked * maskf)
    msk_acc[...] = msk_acc[...] + jnp.sum(maskf)

    ang = _angle(ni, nr)
    cm_acc[...] = cm_acc[...] + jax.lax.dot_general(
        onehot, ang, (((0,), (0,)), ((), ())),
        preferred_element_type=jnp.float32)
    cnt_acc[...] = cnt_acc[...] + jax.lax.dot_general(
        onehot, jnp.ones((bn, h_dim), jnp.float32), (((0,), (0,)), ((), ())),
        preferred_element_type=jnp.float32)

    if final:
        hr = jnp.maximum(nr, 0.0)
        hi = jnp.maximum(ni, 0.0)
        hr_ref[...] = hr
        hi_ref[...] = hi
        ang_ref[...] = _angle(hi, hr)
        nrm_ref[...] = jnp.sqrt(hr * hr + hi * hi + 1e-12)
        oo_ref[...] = jnp.dot(jnp.maximum(n, 0.0), mo_ref[...],
                              preferred_element_type=jnp.float32) + bo_ref[...]
    else:
        hout_ref[...] = jnp.maximum(n, 0.0).astype(jnp.bfloat16)

    @pl.when(i == nblk - 1)
    def _finalize():
        cm = cm_acc[...] / (cnt_acc[...] + 1e-8)
        sc = jnp.sum(jnp.cos(cm), axis=0)
        ss = jnp.sum(jnp.sin(cm), axis=0)
        tot = jnp.sum(sc * sc + ss * ss) - float(n_cls * h_dim)
        sep = tot / float((n_cls * n_cls - n_cls) * h_dim)
        sep_ref[...] = sep * jnp.ones((1, 1), jnp.float32)
        sup = -sup_acc[0, 0] / (msk_acc[0, 0] + 1e-8)
        sup_ref[...] = sup * jnp.ones((1, 1), jnp.float32)


def kernel(x, adj, Wr0, Wi0, br0, bi0, conv_Wr, conv_Wi, conv_Wc,
           Wr1, Wi1, br1, bi1, labels, train_mask):
    n_nodes = adj.shape[0]
    h_dim = Wr0.shape[1]
    n_cls = conv_Wc.shape[-1]
    n_layers = conv_Wr.shape[0]
    bn = next(b for b in (400, 200, 100, 40, 8, 4, 2, 1) if n_nodes % b == 0)
    nblk = n_nodes // bn

    w0 = jnp.concatenate([Wr0, Wi0], axis=1)
    b0 = jnp.concatenate([br0, bi0])[None, :]
    h = pl.pallas_call(
        _lin0_body,
        out_shape=jax.ShapeDtypeStruct((n_nodes, 2 * h_dim), jnp.bfloat16),
    )(x, w0, b0)

    lab2 = labels.reshape(n_nodes, 1)
    msk2 = train_mask.astype(jnp.float32).reshape(n_nodes, 1)

    common_in_specs = [
        pl.BlockSpec((bn, n_nodes), lambda i: (i, 0)),
        pl.BlockSpec((n_nodes, 2 * h_dim), lambda i: (0, 0)),
        pl.BlockSpec((2 * h_dim, 2 * h_dim), lambda i: (0, 0)),
        pl.BlockSpec((2 * h_dim, n_cls), lambda i: (0, 0)),
        pl.BlockSpec((bn, 1), lambda i: (i, 0)),
        pl.BlockSpec((bn, 1), lambda i: (i, 0)),
    ]
    scalar_spec = pl.BlockSpec((1, 1), lambda i: (0, 0))
    scalar_shape = jax.ShapeDtypeStruct((1, 1), jnp.float32)
    scratch = [
        pltpu.VMEM((n_cls, h_dim), jnp.float32),
        pltpu.VMEM((n_cls, h_dim), jnp.float32),
        pltpu.VMEM((1, 1), jnp.float32),
        pltpu.VMEM((1, 1), jnp.float32),
    ]

    seps, sups = [], []
    adj_in = adj
    for l in range(n_layers):
        m = jnp.block([[conv_Wr[l], conv_Wi[l]], [-conv_Wi[l], conv_Wr[l]]])
        wc = conv_Wc[l]
        last = l == n_layers - 1
        if not last:
            body = functools.partial(_layer_body, final=False, bn=bn,
                                     nblk=nblk, h_dim=h_dim, n_cls=n_cls)
            h, adj_in, sep, sup = pl.pallas_call(
                body,
                grid=(nblk,),
                in_specs=common_in_specs,
                out_specs=[
                    pl.BlockSpec((bn, 2 * h_dim), lambda i: (i, 0)),
                    pl.BlockSpec((bn, n_nodes), lambda i: (i, 0)),
                    scalar_spec, scalar_spec,
                ],
                out_shape=[
                    jax.ShapeDtypeStruct((n_nodes, 2 * h_dim), jnp.bfloat16),
                    jax.ShapeDtypeStruct((n_nodes, n_nodes), jnp.bfloat16),
                    scalar_shape, scalar_shape,
                ],
                scratch_shapes=scratch,
            )(adj_in, h, m, wc, lab2, msk2)
        else:
            mo = jnp.block([[Wr1, Wi1], [-Wi1, Wr1]])
            bo = jnp.concatenate([br1, bi1])[None, :]
            body = functools.partial(_layer_body, final=True, bn=bn,
                                     nblk=nblk, h_dim=h_dim, n_cls=n_cls)
            hr, hi, hang, hnrm, oo, sep, sup = pl.pallas_call(
                body,
                grid=(nblk,),
                in_specs=common_in_specs + [
                    pl.BlockSpec((2 * h_dim, 2 * n_cls), lambda i: (0, 0)),
                    pl.BlockSpec((1, 2 * n_cls), lambda i: (0, 0)),
                ],
                out_specs=[
                    pl.BlockSpec((bn, h_dim), lambda i: (i, 0)),
                    pl.BlockSpec((bn, h_dim), lambda i: (i, 0)),
                    pl.BlockSpec((bn, h_dim), lambda i: (i, 0)),
                    pl.BlockSpec((bn, h_dim), lambda i: (i, 0)),
                    pl.BlockSpec((bn, 2 * n_cls), lambda i: (i, 0)),
                    scalar_spec, scalar_spec,
                ],
                out_shape=[
                    jax.ShapeDtypeStruct((n_nodes, h_dim), jnp.float32),
                    jax.ShapeDtypeStruct((n_nodes, h_dim), jnp.float32),
                    jax.ShapeDtypeStruct((n_nodes, h_dim), jnp.float32),
                    jax.ShapeDtypeStruct((n_nodes, h_dim), jnp.float32),
                    jax.ShapeDtypeStruct((n_nodes, 2 * n_cls), jnp.float32),
                    scalar_shape, scalar_shape,
                ],
                scratch_shapes=scratch,
            )(adj_in, h, m, wc, lab2, msk2, mo, bo)
        seps.append(sep)
        sups.append(sup)

    out_r = oo[:, :n_cls]
    out_i = oo[:, n_cls:]
    sep_total = functools.reduce(jnp.add, seps)[0, 0]
    sup_total = functools.reduce(jnp.add, sups)[0, 0]
    return (hr, hi, out_r, out_i, hang, hnrm, sep_total, sup_total)


# speedup vs baseline: 1.1005x; 1.1005x over previous
import functools

import jax
import jax.numpy as jnp
from jax.experimental import pallas as pl
from jax.experimental.pallas import tpu as pltpu


def _angle(hi, hr):
    m = (hr * hr + hi * hi) > 1e-12
    hr_s = jnp.where(m, hr, 1.0)
    hi_s = jnp.where(m, hi, 0.0)
    return jnp.where(m, jnp.arctan2(hi_s, hr_s), 0.0)


def _lin0_body(x_ref, w_ref, b_ref, h_ref):
    h_ref[...] = jnp.maximum(
        jnp.dot(x_ref[...], w_ref[...], preferred_element_type=jnp.float32)
        + b_ref[...], 0.0)


def _layer_body(*refs, final, bn, nblk, h_dim, n_cls, nk):
    if final:
        (adj0_ref, adj1_ref, h_ref, m_ref, wc_ref, lab_ref, msk_ref,
         mo_ref, bo_ref,
         hr_ref, hi_ref, ang_ref, nrm_ref, oo_ref, sep_ref, sup_ref,
         cm_acc, cnt_acc, sup_acc, msk_acc) = refs
    else:
        (adj0_ref, adj1_ref, h_ref, m_ref, wc_ref, lab_ref, msk_ref,
         hout_ref, sep_ref, sup_ref,
         cm_acc, cnt_acc, sup_acc, msk_acc) = refs
    i = pl.program_id(0)

    @pl.when(i == 0)
    def _init():
        cm_acc[...] = jnp.zeros_like(cm_acc)
        cnt_acc[...] = jnp.zeros_like(cnt_acc)
        sup_acc[...] = jnp.zeros_like(sup_acc)
        msk_acc[...] = jnp.zeros_like(msk_acc)

    a = jnp.concatenate(
        [jnp.dot(adj0_ref[...], h_ref[...],
                 preferred_element_type=jnp.float32),
         jnp.dot(adj1_ref[...], h_ref[...],
                 preferred_element_type=jnp.float32)], axis=0)
    n = jnp.dot(a, m_ref[...], preferred_element_type=jnp.float32)
    nr = n[:, :h_dim]
    ni = n[:, h_dim:]

    logits = jnp.dot(n, wc_ref[...], preferred_element_type=jnp.float32)
    mx = jnp.max(logits, axis=-1, keepdims=True)
    lse = mx + jnp.log(jnp.sum(jnp.exp(logits - mx), axis=-1, keepdims=True))
    logp = logits - lse
    lab = lab_ref[...]
    classes = jax.lax.broadcasted_iota(jnp.int32, (bn, n_cls), 1)
    onehot = (lab == classes).astype(jnp.float32)
    maskf = msk_ref[...]
    picked = jnp.sum(onehot * logp, axis=-1, keepdims=True)
    sup_acc[...] = sup_acc[...] + jnp.sum(picked * maskf)
    msk_acc[...] = msk_acc[...] + jnp.sum(maskf)

    ang = _angle(ni, nr)
    cm_acc[...] = cm_acc[...] + jax.lax.dot_general(
        onehot, ang, (((0,), (0,)), ((), ())),
        preferred_element_type=jnp.float32)
    cnt_acc[...] = cnt_acc[...] + jax.lax.dot_general(
        onehot, jnp.ones((bn, h_dim), jnp.float32), (((0,), (0,)), ((), ())),
        preferred_element_type=jnp.float32)

    if final:
        hr = jnp.maximum(nr, 0.0)
        hi = jnp.maximum(ni, 0.0)
        hr_ref[...] = hr
        hi_ref[...] = hi
        ang_ref[...] = _angle(hi, hr)
        nrm_ref[...] = jnp.sqrt(hr * hr + hi * hi + 1e-12)
        oo_ref[...] = jnp.dot(jnp.maximum(n, 0.0), mo_ref[...],
                              preferred_element_type=jnp.float32) + bo_ref[...]
    else:
        hout_ref[...] = jnp.maximum(n, 0.0)

    @pl.when(i == nblk - 1)
    def _finalize():
        cm = cm_acc[...] / (cnt_acc[...] + 1e-8)
        sc = jnp.sum(jnp.cos(cm), axis=0)
        ss = jnp.sum(jnp.sin(cm), axis=0)
        tot = jnp.sum(sc * sc + ss * ss) - float(n_cls * h_dim)
        sep = tot / float((n_cls * n_cls - n_cls) * h_dim)
        sep_ref[...] = sep * jnp.ones((1, 1), jnp.float32)
        sup = -sup_acc[0, 0] / (msk_acc[0, 0] + 1e-8)
        sup_ref[...] = sup * jnp.ones((1, 1), jnp.float32)


def kernel(x, adj, Wr0, Wi0, br0, bi0, conv_Wr, conv_Wi, conv_Wc,
           Wr1, Wi1, br1, bi1, labels, train_mask):
    n_nodes = adj.shape[0]
    h_dim = Wr0.shape[1]
    n_cls = conv_Wc.shape[-1]
    n_layers = conv_Wr.shape[0]
    bn = next(b for b in (400, 200, 100, 40, 8, 4, 2, 1) if n_nodes % b == 0)
    nblk = n_nodes // bn
    nk = n_nodes // 2

    w0 = jnp.concatenate([Wr0, Wi0], axis=1)
    b0 = jnp.concatenate([br0, bi0])[None, :]
    h = pl.pallas_call(
        _lin0_body,
        out_shape=jax.ShapeDtypeStruct((n_nodes, 2 * h_dim), jnp.float32),
    )(x, w0, b0)

    lab2 = labels.reshape(n_nodes, 1)
    msk2 = train_mask.astype(jnp.float32).reshape(n_nodes, 1)

    common_in_specs = [
        pl.BlockSpec((bn // 2, n_nodes), lambda i: (2 * i, 0)),
        pl.BlockSpec((bn // 2, n_nodes), lambda i: (2 * i + 1, 0)),
        pl.BlockSpec((n_nodes, 2 * h_dim), lambda i: (0, 0)),
        pl.BlockSpec((2 * h_dim, 2 * h_dim), lambda i: (0, 0)),
        pl.BlockSpec((2 * h_dim, n_cls), lambda i: (0, 0)),
        pl.BlockSpec((bn, 1), lambda i: (i, 0)),
        pl.BlockSpec((bn, 1), lambda i: (i, 0)),
    ]
    scalar_spec = pl.BlockSpec((1, 1), lambda i: (0, 0))
    scalar_shape = jax.ShapeDtypeStruct((1, 1), jnp.float32)
    scratch = [
        pltpu.VMEM((n_cls, h_dim), jnp.float32),
        pltpu.VMEM((n_cls, h_dim), jnp.float32),
        pltpu.VMEM((1, 1), jnp.float32),
        pltpu.VMEM((1, 1), jnp.float32),
    ]

    seps, sups = [], []
    for l in range(n_layers):
        m = jnp.block([[conv_Wr[l], conv_Wi[l]], [-conv_Wi[l], conv_Wr[l]]])
        wc = conv_Wc[l]
        last = l == n_layers - 1
        if not last:
            body = functools.partial(_layer_body, final=False, bn=bn,
                                     nblk=nblk, h_dim=h_dim, n_cls=n_cls,
                                     nk=nk)
            h, sep, sup = pl.pallas_call(
                body,
                grid=(nblk,),
                in_specs=common_in_specs,
                out_specs=[
                    pl.BlockSpec((bn, 2 * h_dim), lambda i: (i, 0)),
                    scalar_spec, scalar_spec,
                ],
                out_shape=[
                    jax.ShapeDtypeStruct((n_nodes, 2 * h_dim), jnp.float32),
                    scalar_shape, scalar_shape,
                ],
                scratch_shapes=scratch,
            )(adj, adj, h, m, wc, lab2, msk2)
        else:
            mo = jnp.block([[Wr1, Wi1], [-Wi1, Wr1]])
            bo = jnp.concatenate([br1, bi1])[None, :]
            body = functools.partial(_layer_body, final=True, bn=bn,
                                     nblk=nblk, h_dim=h_dim, n_cls=n_cls,
                                     nk=nk)
            hr, hi, hang, hnrm, oo, sep, sup = pl.pallas_call(
                body,
                grid=(nblk,),
                in_specs=common_in_specs + [
                    pl.BlockSpec((2 * h_dim, 2 * n_cls), lambda i: (0, 0)),
                    pl.BlockSpec((1, 2 * n_cls), lambda i: (0, 0)),
                ],
                out_specs=[
                    pl.BlockSpec((bn, h_dim), lambda i: (i, 0)),
                    pl.BlockSpec((bn, h_dim), lambda i: (i, 0)),
                    pl.BlockSpec((bn, h_dim), lambda i: (i, 0)),
                    pl.BlockSpec((bn, h_dim), lambda i: (i, 0)),
                    pl.BlockSpec((bn, 2 * n_cls), lambda i: (i, 0)),
                    scalar_spec, scalar_spec,
                ],
                out_shape=[
                    jax.ShapeDtypeStruct((n_nodes, h_dim), jnp.float32),
                    jax.ShapeDtypeStruct((n_nodes, h_dim), jnp.float32),
                    jax.ShapeDtypeStruct((n_nodes, h_dim), jnp.float32),
                    jax.ShapeDtypeStruct((n_nodes, h_dim), jnp.float32),
                    jax.ShapeDtypeStruct((n_nodes, 2 * n_cls), jnp.float32),
                    scalar_shape, scalar_shape,
                ],
                scratch_shapes=scratch,
            )(adj, adj, h, m, wc, lab2, msk2, mo, bo)
        seps.append(sep)
        sups.append(sup)

    out_r = oo[:, :n_cls]
    out_i = oo[:, n_cls:]
    sep_total = functools.reduce(jnp.add, seps)[0, 0]
    sup_total = functools.reduce(jnp.add, sups)[0, 0]
    return (hr, hi, out_r, out_i, hang, hnrm, sep_total, sup_total)
